# baseline (speedup 1.0000x reference)
import jax
import jax.numpy as jnp
from jax.experimental import pallas as pl
from jax.experimental.pallas import tpu as pltpu

B = 4
S = 8192
D = 4096
E = 64

BLK = 256


def _body(x_ref, w_ref, b_ref, o_ref, acc_ref):
    i = pl.program_id(0)
    nsteps = pl.num_programs(0)

    @pl.when(i == 0)
    def _init():
        acc_ref[...] = jnp.zeros_like(acc_ref)

    acc_ref[...] += jnp.sum(x_ref[...], axis=1)

    @pl.when(i == nsteps - 1)
    def _finish():
        xbar = acc_ref[...] * (1.0 / S)
        scores = jax.lax.dot_general(
            xbar, w_ref[...],
            dimension_numbers=(((1,), (1,)), ((), ())),
            preferred_element_type=jnp.float32,
        ) + b_ref[...]
        m = jnp.max(scores, axis=1, keepdims=True)
        ex = jnp.exp(scores - m)
        o_ref[...] = ex / jnp.sum(ex, axis=1, keepdims=True)


def kernel(x, W, b):
    b2 = b.reshape(1, E)
    grid = (S // BLK,)
    return pl.pallas_call(
        _body,
        grid=grid,
        in_specs=[
            pl.BlockSpec((B, BLK, D), lambda i: (0, i, 0)),
            pl.BlockSpec((E, D), lambda i: (0, 0)),
            pl.BlockSpec((1, E), lambda i: (0, 0)),
        ],
        out_specs=pl.BlockSpec((B, E), lambda i: (0, 0)),
        out_shape=jax.ShapeDtypeStruct((B, E), jnp.float32),
        scratch_shapes=[pltpu.VMEM((B, D), jnp.float32)],
    )(x, W, b2)


# speedup vs baseline: 1.1269x; 1.1269x over previous
import jax
import jax.numpy as jnp
from jax.experimental import pallas as pl
from jax.experimental.pallas import tpu as pltpu

B = 4
S = 8192
D = 4096
E = 64

BLKR = 1024


def _body(x_ref, w_ref, b_ref, o_ref, acc_ref):
    i = pl.program_id(0)
    nsteps = pl.num_programs(0)

    @pl.when(i == 0)
    def _init():
        acc_ref[...] = jnp.zeros_like(acc_ref)

    partial = jnp.sum(x_ref[...], axis=0, keepdims=True)
    bidx = i // (S // BLKR)
    onehot = jax.lax.broadcasted_iota(jnp.int32, (B, 1), 0) == bidx
    acc_ref[...] += jnp.where(onehot, partial, 0.0)

    @pl.when(i == nsteps - 1)
    def _finish():
        xbar = acc_ref[...] * (1.0 / S)
        scores = jax.lax.dot_general(
            xbar, w_ref[...],
            dimension_numbers=(((1,), (1,)), ((), ())),
            preferred_element_type=jnp.float32,
        ) + b_ref[...]
        m = jnp.max(scores, axis=1, keepdims=True)
        ex = jnp.exp(scores - m)
        o_ref[...] = ex / jnp.sum(ex, axis=1, keepdims=True)


def kernel(x, W, b):
    xf = x.reshape(B * S, D)
    b2 = b.reshape(1, E)
    grid = (B * S // BLKR,)
    return pl.pallas_call(
        _body,
        grid=grid,
        in_specs=[
            pl.BlockSpec((BLKR, D), lambda i: (i, 0)),
            pl.BlockSpec((E, D), lambda i: (0, 0)),
            pl.BlockSpec((1, E), lambda i: (0, 0)),
        ],
        out_specs=pl.BlockSpec((B, E), lambda i: (0, 0)),
        out_shape=jax.ShapeDtypeStruct((B, E), jnp.float32),
        scratch_shapes=[pltpu.VMEM((B, D), jnp.float32)],
    )(xf, W, b2)
